# baseline (device time: 63221 ns/iter reference)
import jax
import jax.numpy as jnp
from jax import lax
from jax.experimental import pallas as pl
from jax.experimental.pallas import tpu as pltpu


def kernel(x, dy):
    k, m = x.shape
    _, f = dy.shape
    fh = f // 2
    mh = m // 2

    NC = 16
    fc = fh // NC
    BLOCKS = [1, 3, 4, 4, 4]
    NB = len(BLOCKS)
    STARTS = [sum(BLOCKS[:i]) for i in range(NB)]
    dims = (((1,), (0,)), ((), ()))

    def body(x_ref, dy_ref, out_ref, dyv_ref, ps_ref, pm_ref, rx_ref,
             ov_ref, dysems, osems, sxs, rxs, sys_, rys):
        my_x = lax.axis_index("x")
        my_y = lax.axis_index("y")

        dy_copies = []
        for b in range(NB):
            s, w = STARTS[b] * fc, BLOCKS[b] * fc
            cp = pltpu.make_async_copy(
                dy_ref.at[:, pl.ds(my_y * fh + s, w)],
                dyv_ref.at[:, pl.ds(s, w)],
                dysems.at[b],
            )
            cp.start()
            dy_copies.append(cp)

        barrier = pltpu.get_barrier_semaphore()
        pl.semaphore_signal(
            barrier, inc=1, device_id=(1 - my_x, my_y),
            device_id_type=pl.DeviceIdType.MESH,
        )
        pl.semaphore_signal(
            barrier, inc=1, device_id=(my_x, 1 - my_y),
            device_id_type=pl.DeviceIdType.MESH,
        )

        xo = x_ref[:, pl.ds((1 - my_x) * mh, mh)].T
        xm = x_ref[:, pl.ds(my_x * mh, mh)].T

        pl.semaphore_wait(barrier, 2)

        rdmas_x = []
        for b in range(NB):
            dy_copies[b].wait()
            s, w = STARTS[b] * fc, BLOCKS[b] * fc
            ps_ref[:, pl.ds(s, w)] = lax.dot_general(
                xo, dyv_ref[:, pl.ds(s, w)], dims,
                preferred_element_type=jnp.float32,
            )
            for i in range(BLOCKS[b]):
                c = STARTS[b] + i
                r = pltpu.make_async_remote_copy(
                    src_ref=ps_ref.at[:, pl.ds(c * fc, fc)],
                    dst_ref=rx_ref.at[:, pl.ds(c * fc, fc)],
                    send_sem=sxs.at[c],
                    recv_sem=rxs.at[c],
                    device_id=(1 - my_x, my_y),
                    device_id_type=pl.DeviceIdType.MESH,
                )
                r.start()
                rdmas_x.append(r)

        for b in range(NB):
            s, w = STARTS[b] * fc, BLOCKS[b] * fc
            pm_ref[:, pl.ds(s, w)] = lax.dot_general(
                xm, dyv_ref[:, pl.ds(s, w)], dims,
                preferred_element_type=jnp.float32,
            )

        rdmas_y = []
        out_copies = []
        for c in range(NC):
            rdmas_x[c].wait_recv()
            ov_ref[:, pl.ds(c * fc, fc)] = (
                pm_ref[:, pl.ds(c * fc, fc)] + rx_ref[:, pl.ds(c * fc, fc)]
            )
            oc = pltpu.make_async_copy(
                ov_ref.at[:, pl.ds(c * fc, fc)],
                out_ref.at[:, pl.ds(my_y * fh + c * fc, fc)],
                osems.at[c],
            )
            oc.start()
            out_copies.append(oc)
            r = pltpu.make_async_remote_copy(
                src_ref=ov_ref.at[:, pl.ds(c * fc, fc)],
                dst_ref=out_ref.at[:, pl.ds(my_y * fh + c * fc, fc)],
                send_sem=sys_.at[c],
                recv_sem=rys.at[c],
                device_id=(my_x, 1 - my_y),
                device_id_type=pl.DeviceIdType.MESH,
            )
            r.start()
            rdmas_y.append(r)

        for c in range(NC):
            rdmas_y[c].wait_recv()
            rdmas_y[c].wait_send()
            rdmas_x[c].wait_send()
            out_copies[c].wait()

    return pl.pallas_call(
        body,
        out_shape=jax.ShapeDtypeStruct((mh, f), jnp.float32),
        in_specs=[
            pl.BlockSpec(memory_space=pltpu.VMEM),
            pl.BlockSpec(memory_space=pltpu.HBM),
        ],
        out_specs=pl.BlockSpec(memory_space=pltpu.HBM),
        scratch_shapes=[
            pltpu.VMEM((k, fh), jnp.float32),
            pltpu.VMEM((mh, fh), jnp.float32),
            pltpu.VMEM((mh, fh), jnp.float32),
            pltpu.VMEM((mh, fh), jnp.float32),
            pltpu.VMEM((mh, fh), jnp.float32),
            pltpu.SemaphoreType.DMA((NB,)),
            pltpu.SemaphoreType.DMA((NC,)),
            pltpu.SemaphoreType.DMA((NC,)),
            pltpu.SemaphoreType.DMA((NC,)),
            pltpu.SemaphoreType.DMA((NC,)),
            pltpu.SemaphoreType.DMA((NC,)),
        ],
        compiler_params=pltpu.CompilerParams(collective_id=0),
    )(x, dy)


# device time: 63182 ns/iter; 1.0006x vs baseline; 1.0006x over previous
import jax
import jax.numpy as jnp
from jax import lax
from jax.experimental import pallas as pl
from jax.experimental.pallas import tpu as pltpu


def kernel(x, dy):
    k, m = x.shape
    _, f = dy.shape
    fh = f // 2
    mh = m // 2

    NC = 16
    fc = fh // NC
    BLOCKS = [1, 3, 4, 4, 4]
    NB = len(BLOCKS)
    STARTS = [sum(BLOCKS[:i]) for i in range(NB)]
    dims = (((1,), (0,)), ((), ()))

    def body(x_ref, dy_ref, out_ref, dyv_ref, ps_ref, pm_ref, rx_ref,
             dysems, sxs, rxs, sys_, rys):
        my_x = lax.axis_index("x")
        my_y = lax.axis_index("y")

        dy_copies = []
        for b in range(NB):
            s, w = STARTS[b] * fc, BLOCKS[b] * fc
            cp = pltpu.make_async_copy(
                dy_ref.at[:, pl.ds(my_y * fh + s, w)],
                dyv_ref.at[:, pl.ds(s, w)],
                dysems.at[b],
            )
            cp.start()
            dy_copies.append(cp)

        barrier = pltpu.get_barrier_semaphore()
        pl.semaphore_signal(
            barrier, inc=1, device_id=(1 - my_x, my_y),
            device_id_type=pl.DeviceIdType.MESH,
        )
        pl.semaphore_signal(
            barrier, inc=1, device_id=(my_x, 1 - my_y),
            device_id_type=pl.DeviceIdType.MESH,
        )

        xo = x_ref[:, pl.ds((1 - my_x) * mh, mh)].T
        xm = x_ref[:, pl.ds(my_x * mh, mh)].T

        pl.semaphore_wait(barrier, 2)

        rdmas_x = []
        for b in range(NB):
            dy_copies[b].wait()
            s, w = STARTS[b] * fc, BLOCKS[b] * fc
            ps_ref[:, pl.ds(s, w)] = lax.dot_general(
                xo, dyv_ref[:, pl.ds(s, w)], dims,
                preferred_element_type=jnp.float32,
            )
            for i in range(BLOCKS[b]):
                c = STARTS[b] + i
                r = pltpu.make_async_remote_copy(
                    src_ref=ps_ref.at[:, pl.ds(c * fc, fc)],
                    dst_ref=rx_ref.at[:, pl.ds(c * fc, fc)],
                    send_sem=sxs.at[c],
                    recv_sem=rxs.at[c],
                    device_id=(1 - my_x, my_y),
                    device_id_type=pl.DeviceIdType.MESH,
                )
                r.start()
                rdmas_x.append(r)

        for b in range(NB):
            s, w = STARTS[b] * fc, BLOCKS[b] * fc
            pm_ref[:, pl.ds(s, w)] = lax.dot_general(
                xm, dyv_ref[:, pl.ds(s, w)], dims,
                preferred_element_type=jnp.float32,
            )

        rdmas_y = []
        for c in range(NC):
            rdmas_x[c].wait_recv()
            out_ref[:, pl.ds(my_y * fh + c * fc, fc)] = (
                pm_ref[:, pl.ds(c * fc, fc)] + rx_ref[:, pl.ds(c * fc, fc)]
            )
            r = pltpu.make_async_remote_copy(
                src_ref=out_ref.at[:, pl.ds(my_y * fh + c * fc, fc)],
                dst_ref=out_ref.at[:, pl.ds(my_y * fh + c * fc, fc)],
                send_sem=sys_.at[c],
                recv_sem=rys.at[c],
                device_id=(my_x, 1 - my_y),
                device_id_type=pl.DeviceIdType.MESH,
            )
            r.start()
            rdmas_y.append(r)

        for c in range(NC):
            rdmas_y[c].wait_recv()
            rdmas_y[c].wait_send()
            rdmas_x[c].wait_send()

    return pl.pallas_call(
        body,
        out_shape=jax.ShapeDtypeStruct((mh, f), jnp.float32),
        in_specs=[
            pl.BlockSpec(memory_space=pltpu.VMEM),
            pl.BlockSpec(memory_space=pltpu.HBM),
        ],
        out_specs=pl.BlockSpec(memory_space=pltpu.VMEM),
        scratch_shapes=[
            pltpu.VMEM((k, fh), jnp.float32),
            pltpu.VMEM((mh, fh), jnp.float32),
            pltpu.VMEM((mh, fh), jnp.float32),
            pltpu.VMEM((mh, fh), jnp.float32),
            pltpu.SemaphoreType.DMA((NB,)),
            pltpu.SemaphoreType.DMA((NC,)),
            pltpu.SemaphoreType.DMA((NC,)),
            pltpu.SemaphoreType.DMA((NC,)),
            pltpu.SemaphoreType.DMA((NC,)),
        ],
        compiler_params=pltpu.CompilerParams(collective_id=0),
    )(x, dy)


# device time: 62429 ns/iter; 1.0127x vs baseline; 1.0121x over previous
import jax
import jax.numpy as jnp
from jax import lax
from jax.experimental import pallas as pl
from jax.experimental.pallas import tpu as pltpu


def kernel(x, dy):
    k, m = x.shape
    _, f = dy.shape
    fh = f // 2
    mh = m // 2

    NC = 16
    fc = fh // NC
    BLOCKS = [4, 4, 4, 4]
    NB = len(BLOCKS)
    STARTS = [sum(BLOCKS[:i]) for i in range(NB)]
    dims = (((1,), (0,)), ((), ()))

    def body(x_ref, dy_ref, out_ref, dyv_ref, ps_ref, pm_ref, rx_ref,
             dysems, sxs, rxs, sys_, rys):
        my_x = lax.axis_index("x")
        my_y = lax.axis_index("y")

        scope = jax.named_scope

        dy_copies = []
        for b in range(NB):
            s, w = STARTS[b] * fc, BLOCKS[b] * fc
            cp = pltpu.make_async_copy(
                dy_ref.at[:, pl.ds(my_y * fh + s, w)],
                dyv_ref.at[:, pl.ds(s, w)],
                dysems.at[b],
            )
            cp.start()
            dy_copies.append(cp)

        barrier = pltpu.get_barrier_semaphore()
        pl.semaphore_signal(
            barrier, inc=1, device_id=(1 - my_x, my_y),
            device_id_type=pl.DeviceIdType.MESH,
        )
        pl.semaphore_signal(
            barrier, inc=1, device_id=(my_x, 1 - my_y),
            device_id_type=pl.DeviceIdType.MESH,
        )

        with scope("transpose"):
            xo = x_ref[:, pl.ds((1 - my_x) * mh, mh)].T
            xm = x_ref[:, pl.ds(my_x * mh, mh)].T

        with scope("barrier_wait"):
            pl.semaphore_wait(barrier, 2)

        rdmas_x = []
        for b in range(NB):
            with scope(f"p1_dot#b={b}"):
                dy_copies[b].wait()
                s, w = STARTS[b] * fc, BLOCKS[b] * fc
                ps_ref[:, pl.ds(s, w)] = lax.dot_general(
                    xo, dyv_ref[:, pl.ds(s, w)], dims,
                    preferred_element_type=jnp.float32,
                )
            with scope(f"p1_send#b={b}"):
                for i in range(BLOCKS[b]):
                    c = STARTS[b] + i
                    r = pltpu.make_async_remote_copy(
                        src_ref=ps_ref.at[:, pl.ds(c * fc, fc)],
                        dst_ref=rx_ref.at[:, pl.ds(c * fc, fc)],
                        send_sem=sxs.at[c],
                        recv_sem=rxs.at[c],
                        device_id=(1 - my_x, my_y),
                        device_id_type=pl.DeviceIdType.MESH,
                    )
                    r.start()
                    rdmas_x.append(r)

        with scope("p15_dots"):
            for b in range(NB):
                s, w = STARTS[b] * fc, BLOCKS[b] * fc
                pm_ref[:, pl.ds(s, w)] = lax.dot_general(
                    xm, dyv_ref[:, pl.ds(s, w)], dims,
                    preferred_element_type=jnp.float32,
                )

        rdmas_y = []
        for c in range(NC):
            with scope(f"p2_wait#c={c}"):
                rdmas_x[c].wait_recv()
            with scope(f"p2_add#c={c}"):
                out_ref[:, pl.ds(my_y * fh + c * fc, fc)] = (
                    pm_ref[:, pl.ds(c * fc, fc)]
                    + rx_ref[:, pl.ds(c * fc, fc)]
                )
                r = pltpu.make_async_remote_copy(
                    src_ref=out_ref.at[:, pl.ds(my_y * fh + c * fc, fc)],
                    dst_ref=out_ref.at[:, pl.ds(my_y * fh + c * fc, fc)],
                    send_sem=sys_.at[c],
                    recv_sem=rys.at[c],
                    device_id=(my_x, 1 - my_y),
                    device_id_type=pl.DeviceIdType.MESH,
                )
                r.start()
                rdmas_y.append(r)

        with scope("drain"):
            for c in range(NC):
                rdmas_y[c].wait_recv()
                rdmas_y[c].wait_send()
                rdmas_x[c].wait_send()

    return pl.pallas_call(
        body,
        out_shape=jax.ShapeDtypeStruct((mh, f), jnp.float32),
        in_specs=[
            pl.BlockSpec(memory_space=pltpu.VMEM),
            pl.BlockSpec(memory_space=pltpu.HBM),
        ],
        out_specs=pl.BlockSpec(memory_space=pltpu.VMEM),
        scratch_shapes=[
            pltpu.VMEM((k, fh), jnp.float32),
            pltpu.VMEM((mh, fh), jnp.float32),
            pltpu.VMEM((mh, fh), jnp.float32),
            pltpu.VMEM((mh, fh), jnp.float32),
            pltpu.SemaphoreType.DMA((NB,)),
            pltpu.SemaphoreType.DMA((NC,)),
            pltpu.SemaphoreType.DMA((NC,)),
            pltpu.SemaphoreType.DMA((NC,)),
            pltpu.SemaphoreType.DMA((NC,)),
        ],
        compiler_params=pltpu.CompilerParams(collective_id=0),
    )(x, dy)
